# baseline (speedup 1.0000x reference)
import functools

import jax
import jax.numpy as jnp
from jax import lax
from jax.experimental import pallas as pl
from jax.experimental.pallas import tpu as pltpu
from jax.experimental.pallas import tpu_sc as plsc

_N = 10000
_E = 320000
_DIN = 128
_DH = 128
_DOUT = 64
_NP = 10240
_EPAD = 327680
_C = 128
_ROWS_PER_TILE = _NP // 16
_DEG_CHUNKS = _EPAD // (32 * _C)
_SC_CHUNKS = _EPAD // (16 * _C)

_mesh = plsc.VectorSubcoreMesh(core_axis_name="c", subcore_axis_name="s")


@functools.partial(
    pl.kernel,
    mesh=_mesh,
    out_type=jax.ShapeDtypeStruct((2, _NP), jnp.float32),
    scratch_types=[
        pltpu.VMEM((_DEG_CHUNKS, _C), jnp.int32),
        pltpu.VMEM((_DEG_CHUNKS, _C), jnp.float32),
        pltpu.VMEM((_ROWS_PER_TILE,), jnp.float32),
        pltpu.VMEM_SHARED((_NP,), jnp.float32),
        pltpu.SemaphoreType.DMA,
    ],
    compiler_params=pltpu.CompilerParams(use_tc_tiling_on_sc=False),
)
def _deg_kernel(col_hbm, ew_hbm, out_hbm, col_v, ew_v, zbuf, acc, sem):
    c = lax.axis_index("c")
    s = lax.axis_index("s")
    wid = s * 2 + c

    def _zero(i, carry):
        zbuf[pl.ds(i * 16, 16)] = jnp.zeros((16,), jnp.float32)
        return carry

    lax.fori_loop(0, _ROWS_PER_TILE // 16, _zero, 0)
    pltpu.sync_copy(zbuf, acc.at[pl.ds(s * _ROWS_PER_TILE, _ROWS_PER_TILE)])
    plsc.subcore_barrier()

    base = wid * _DEG_CHUNKS
    pltpu.sync_copy(col_hbm.at[pl.ds(base, _DEG_CHUNKS)], col_v)
    pltpu.sync_copy(ew_hbm.at[pl.ds(base, _DEG_CHUNKS)], ew_v)

    def _chunk(i, carry):
        pltpu.sync_copy(ew_v.at[i], acc.at[col_v.at[i]], add=True)
        return carry

    lax.fori_loop(0, _DEG_CHUNKS, _chunk, 0)
    plsc.subcore_barrier()
    pltpu.sync_copy(acc.at[pl.ds(s * _ROWS_PER_TILE, _ROWS_PER_TILE)],
                    out_hbm.at[c, pl.ds(s * _ROWS_PER_TILE, _ROWS_PER_TILE)])


def _make_scatter_kernel(D, n_tabs):
    ept = _EPAD // 16
    nbuf = 5
    nhalf = 2
    hchunks = _SC_CHUNKS // nhalf
    hedges = ept // nhalf

    @functools.partial(
        pl.kernel,
        mesh=_mesh,
        out_type=[jax.ShapeDtypeStruct((2, _NP, D), jnp.float32)] * n_tabs,
        scratch_types=[
            pltpu.VMEM((hedges,), jnp.int32),
            pltpu.VMEM((hchunks, _C), jnp.int32),
            pltpu.VMEM((hedges,), jnp.float32),
            pltpu.VMEM((nbuf, _C, D), jnp.float32),
            pltpu.VMEM((_C, D), jnp.float32),
            pltpu.VMEM_SHARED((_NP, D), jnp.float32),
            pltpu.SemaphoreType.DMA,
            pltpu.SemaphoreType.DMA,
        ],
        compiler_params=pltpu.CompilerParams(use_tc_tiling_on_sc=False),
    )
    def _scatter(*refs):
        row_hbm, col_hbm, ew_hbm = refs[0:3]
        tabs = refs[3:3 + n_tabs]
        outs = refs[3 + n_tabs:3 + 2 * n_tabs]
        (row_v, col_v, ew_v, bufs, zbuf, acc, gsem,
         ssem) = refs[3 + 2 * n_tabs:]
        c = lax.axis_index("c")
        s = lax.axis_index("s")

        def _zero(i, carry):
            for j in range(D // 16):
                zbuf[i, pl.ds(j * 16, 16)] = jnp.zeros((16,), jnp.float32)
            return carry

        lax.fori_loop(0, _C, _zero, 0)

        def _drain(sem_):
            pltpu.make_async_copy(tabs[0].at[pl.ds(0, _C)], zbuf, sem_).wait()

        def _scale(i, buf):
            def _group(g, gcarry):
                ew16 = ew_v[pl.ds(i * _C + g * 16, 16)]
                e0 = g * 16
                for e16 in range(16):
                    splat = lax.gather(
                        ew16, jnp.full((16, 1), e16, jnp.int32),
                        lax.GatherDimensionNumbers(
                            offset_dims=(), collapsed_slice_dims=(0,),
                            start_index_map=(0,)),
                        slice_sizes=(1,),
                        mode=lax.GatherScatterMode.PROMISE_IN_BOUNDS)
                    e = e0 + e16
                    for j in range(D // 16):
                        buf[e, pl.ds(j * 16, 16)] = (
                            buf[e, pl.ds(j * 16, 16)] * splat)
                return gcarry

            lax.fori_loop(0, _C // 16, _group, 0, unroll=4)

        def _one_pass(tab_hbm, out_hbm):
            for q in range(_ROWS_PER_TILE // _C):
                pltpu.sync_copy(
                    zbuf, acc.at[pl.ds(s * _ROWS_PER_TILE + q * _C, _C)])
            plsc.subcore_barrier()

            for h in range(nhalf):
                pltpu.sync_copy(
                    row_hbm.at[c, pl.ds(s * ept + h * hedges, hedges)], row_v)
                pltpu.sync_copy(
                    col_hbm.at[pl.ds(s * _SC_CHUNKS + h * hchunks, hchunks)],
                    col_v)
                pltpu.sync_copy(
                    ew_hbm.at[pl.ds(s * ept + h * hedges, hedges)], ew_v)

                def _issue_gather(i, b, tab=tab_hbm):
                    pltpu.async_copy(
                        tab.at[row_v.at[pl.ds(i * _C, _C)]], bufs.at[b], gsem)

                _issue_gather(0, 0)
                _issue_gather(1, 1)

                def _roll(sp, carry, issue=_issue_gather):
                    for b in range(nbuf):
                        i = sp * nbuf + b
                        _drain(gsem)
                        _scale(i, bufs.at[b])
                        pltpu.async_copy(
                            bufs.at[b], acc.at[col_v.at[i]], ssem, add=True)

                        @pl.when(i >= 3)
                        def _():
                            _drain(ssem)

                        @pl.when(i + 2 < hchunks)
                        def _():
                            issue(i + 2, (b + 2) % nbuf)
                    return carry

                lax.fori_loop(0, hchunks // nbuf, _roll, 0)
                _drain(ssem)
                _drain(ssem)
                _drain(ssem)
            plsc.subcore_barrier()
            pltpu.sync_copy(
                acc.at[pl.ds(s * _ROWS_PER_TILE, _ROWS_PER_TILE)],
                out_hbm.at[c, pl.ds(s * _ROWS_PER_TILE, _ROWS_PER_TILE)])

        for tab_hbm, out_hbm in zip(tabs, outs):
            _one_pass(tab_hbm, out_hbm)

    return _scatter


_scatter_dual = _make_scatter_kernel(_DOUT, 2)
_scatter_single = _make_scatter_kernel(_DOUT, 1)


_BM = 2000
_NB = _N // _BM


def _dis_from(degblk):
    deg = degblk[:, 0:1] + degblk[:, 1:2] + 1.0
    return jnp.where(deg > 0, lax.rsqrt(deg), 0.0)


def _xw_body(x_ref, w_ref, deg_ref, ta_ref, tb_ref):
    dis = _dis_from(deg_ref[...])
    t = jnp.dot(x_ref[...], w_ref[...],
                preferred_element_type=jnp.float32) * dis
    ta_ref[...] = t[:, :_DOUT]
    tb_ref[...] = t[:, _DOUT:]


def _layer1_body(tmpa_ref, tmpb_ref, ta_ref, tb_ref, deg_ref, b_ref, a_ref,
                 w2_ref, t2_ref):
    dis = _dis_from(deg_ref[...])
    u = jnp.concatenate(
        [tmpa_ref[0] + ta_ref[...], tmpb_ref[0] + tb_ref[...]],
        axis=1) * dis + b_ref[...]
    u = jnp.where(u >= 0, u, a_ref[...] * u)
    t2_ref[...] = jnp.dot(u, w2_ref[...],
                          preferred_element_type=jnp.float32) * dis


def _layer2_body(tmp_t, tmp_b, t2_t, t2_b, deg_ref, b_ref, a_ref, wd_ref,
                 z_ref, cc_ref, p_ref, pa_ref):
    dis = _dis_from(deg_ref[...])
    zt = (tmp_t[0] + t2_t[...]) * dis + b_ref[...]
    zt = jnp.where(zt >= 0, zt, a_ref[...] * zt)
    zb = (tmp_b[0] + t2_b[...]) * dis + b_ref[...]
    zb = jnp.where(zb >= 0, zb, a_ref[...] * zb)
    z_ref[...] = zt
    cc_ref[...] = jnp.concatenate([zt, zb], axis=1)
    p_ref[...] = jnp.dot(zt, wd_ref[...], preferred_element_type=jnp.float32)
    pa_ref[...] = jnp.dot(zb, wd_ref[...], preferred_element_type=jnp.float32)


_BM5 = 512
_BK5 = 2048
_NK5 = (_N + _BK5 - 1) // _BK5


def _readout_body(mask_ref, cc_ref, p_ref, pa_ref, bd_ref,
                  ret_ref, reta_ref, g_acc, rs_acc):
    k = pl.program_id(1)
    valid = _N - k * _BK5
    m = jnp.where(
        lax.broadcasted_iota(jnp.int32, (_BM5, _BK5), 1) < valid,
        mask_ref[...], 0.0)
    cc = jnp.where(
        lax.broadcasted_iota(jnp.int32, (_BK5, _DH), 0) < valid,
        cc_ref[...], 0.0)
    part = jnp.dot(m, cc, preferred_element_type=jnp.float32)
    rpart = jnp.sum(m, axis=1, keepdims=True)

    @pl.when(k == 0)
    def _init():
        g_acc[...] = part
        rs_acc[...] = rpart

    @pl.when(k > 0)
    def _accum():
        g_acc[...] = g_acc[...] + part
        rs_acc[...] = rs_acc[...] + rpart

    @pl.when(k == _NK5 - 1)
    def _epilogue():
        avg = g_acc[...] / rs_acc[...]
        ga = avg[:, :_DOUT]
        gb = avg[:, _DOUT:]
        na = jnp.maximum(jnp.sqrt(jnp.sum(ga * ga, axis=1, keepdims=True)),
                         1e-12)
        nb = jnp.maximum(jnp.sqrt(jnp.sum(gb * gb, axis=1, keepdims=True)),
                         1e-12)
        gsa = jax.nn.sigmoid(ga / na)
        gsb = jax.nn.sigmoid(gb / nb)
        p = p_ref[...]
        pa = pa_ref[...]
        bd = bd_ref[0, 0]
        ret_ref[...] = jnp.concatenate(
            [jnp.sum(p * gsa, axis=1, keepdims=True),
             jnp.sum(pa * gsa, axis=1, keepdims=True)], axis=1) + bd
        reta_ref[...] = jnp.concatenate(
            [jnp.sum(pa * gsb, axis=1, keepdims=True),
             jnp.sum(p * gsb, axis=1, keepdims=True)], axis=1) + bd


def kernel(expr, expr_a, edge_list, edge_weight, graph_neigh,
           W1, b1, alpha1, W2, b2, alpha2, Wd, bd):
    f32 = jnp.float32
    row = edge_list[0]
    col = edge_list[1]
    pad = _EPAD - _E
    row_p = jnp.concatenate([row, jnp.zeros((pad,), row.dtype)]).astype(jnp.int32)
    col_p = jnp.concatenate([col, jnp.zeros((pad,), col.dtype)]).astype(jnp.int32)
    ew_p = jnp.concatenate([edge_weight, jnp.zeros((pad,), f32)])
    col2d = col_p.reshape(-1, _C)
    ew2d = ew_p.reshape(-1, _C)
    row2 = jnp.stack([row_p, row_p + _N])

    deg_raw = _deg_kernel(col2d, ew2d)
    degpt = deg_raw.T

    x2 = jnp.concatenate([expr, expr_a], axis=0)

    t1a, t1b = pl.pallas_call(
        _xw_body,
        grid=(2 * _NB,),
        in_specs=[
            pl.BlockSpec((_BM, _DIN), lambda i: (i, 0)),
            pl.BlockSpec((_DIN, _DH), lambda i: (0, 0)),
            pl.BlockSpec((_BM, 2), lambda i: (i % _NB, 0)),
        ],
        out_specs=[
            pl.BlockSpec((_BM, _DOUT), lambda i: (i, 0)),
            pl.BlockSpec((_BM, _DOUT), lambda i: (i, 0)),
        ],
        out_shape=[
            jax.ShapeDtypeStruct((2 * _N, _DOUT), f32),
            jax.ShapeDtypeStruct((2 * _N, _DOUT), f32),
        ],
        compiler_params=pltpu.CompilerParams(
            dimension_semantics=("parallel",)),
    )(x2, W1, degpt)

    tmp1a, tmp1b = _scatter_dual(row2, col2d, ew_p, t1a, t1b)

    t2 = pl.pallas_call(
        _layer1_body,
        grid=(2 * _NB,),
        in_specs=[
            pl.BlockSpec((1, _BM, _DOUT), lambda i: (i // _NB, i % _NB, 0)),
            pl.BlockSpec((1, _BM, _DOUT), lambda i: (i // _NB, i % _NB, 0)),
            pl.BlockSpec((_BM, _DOUT), lambda i: (i, 0)),
            pl.BlockSpec((_BM, _DOUT), lambda i: (i, 0)),
            pl.BlockSpec((_BM, 2), lambda i: (i % _NB, 0)),
            pl.BlockSpec((1, _DH), lambda i: (0, 0)),
            pl.BlockSpec((1, _DH), lambda i: (0, 0)),
            pl.BlockSpec((_DH, _DOUT), lambda i: (0, 0)),
        ],
        out_specs=pl.BlockSpec((_BM, _DOUT), lambda i: (i, 0)),
        out_shape=jax.ShapeDtypeStruct((2 * _N, _DOUT), f32),
        compiler_params=pltpu.CompilerParams(
            dimension_semantics=("parallel",)),
    )(tmp1a, tmp1b, t1a, t1b, degpt, b1.reshape(1, _DH),
      alpha1.reshape(1, _DH), W2)

    tmp2, = _scatter_single(row2, col2d, ew_p, t2)

    z, cc, p, pa = pl.pallas_call(
        _layer2_body,
        grid=(_NB,),
        in_specs=[
            pl.BlockSpec((1, _BM, _DOUT), lambda i: (0, i, 0)),
            pl.BlockSpec((1, _BM, _DOUT), lambda i: (1, i, 0)),
            pl.BlockSpec((_BM, _DOUT), lambda i: (i, 0)),
            pl.BlockSpec((_BM, _DOUT), lambda i: (i + _NB, 0)),
            pl.BlockSpec((_BM, 2), lambda i: (i, 0)),
            pl.BlockSpec((1, _DOUT), lambda i: (0, 0)),
            pl.BlockSpec((1, _DOUT), lambda i: (0, 0)),
            pl.BlockSpec((_DOUT, _DOUT), lambda i: (0, 0)),
        ],
        out_specs=[
            pl.BlockSpec((_BM, _DOUT), lambda i: (i, 0)),
            pl.BlockSpec((_BM, _DH), lambda i: (i, 0)),
            pl.BlockSpec((_BM, _DOUT), lambda i: (i, 0)),
            pl.BlockSpec((_BM, _DOUT), lambda i: (i, 0)),
        ],
        out_shape=[
            jax.ShapeDtypeStruct((_N, _DOUT), f32),
            jax.ShapeDtypeStruct((_N, _DH), f32),
            jax.ShapeDtypeStruct((_N, _DOUT), f32),
            jax.ShapeDtypeStruct((_N, _DOUT), f32),
        ],
        compiler_params=pltpu.CompilerParams(
            dimension_semantics=("parallel",)),
    )(tmp2, tmp2, t2, t2, degpt, b2.reshape(1, _DOUT),
      alpha2.reshape(1, _DOUT), Wd)

    ret, ret_a = pl.pallas_call(
        _readout_body,
        grid=((_N + _BM5 - 1) // _BM5, _NK5),
        in_specs=[
            pl.BlockSpec((_BM5, _BK5), lambda i, k: (i, k)),
            pl.BlockSpec((_BK5, _DH), lambda i, k: (k, 0)),
            pl.BlockSpec((_BM5, _DOUT), lambda i, k: (i, 0)),
            pl.BlockSpec((_BM5, _DOUT), lambda i, k: (i, 0)),
            pl.BlockSpec((1, 1), lambda i, k: (0, 0)),
        ],
        out_specs=[
            pl.BlockSpec((_BM5, 2), lambda i, k: (i, 0)),
            pl.BlockSpec((_BM5, 2), lambda i, k: (i, 0)),
        ],
        out_shape=[
            jax.ShapeDtypeStruct((_N, 2), f32),
            jax.ShapeDtypeStruct((_N, 2), f32),
        ],
        scratch_shapes=[
            pltpu.VMEM((_BM5, _DH), f32),
            pltpu.VMEM((_BM5, 1), f32),
        ],
        compiler_params=pltpu.CompilerParams(
            dimension_semantics=("parallel", "arbitrary")),
    )(graph_neigh, cc, p, pa, bd.reshape(1, 1))

    return (z, ret, ret_a)


# speedup vs baseline: 1.0472x; 1.0472x over previous
import functools

import jax
import jax.numpy as jnp
from jax import lax
from jax.experimental import pallas as pl
from jax.experimental.pallas import tpu as pltpu
from jax.experimental.pallas import tpu_sc as plsc

_N = 10000
_E = 320000
_DIN = 128
_DH = 128
_DOUT = 64
_NP = 10240
_EPAD = 327680
_C = 128
_ROWS_PER_TILE = _NP // 16
_DEG_CHUNKS = _EPAD // (32 * _C)
_SC_CHUNKS = _EPAD // (16 * _C)

_mesh = plsc.VectorSubcoreMesh(core_axis_name="c", subcore_axis_name="s")


@functools.partial(
    pl.kernel,
    mesh=_mesh,
    out_type=jax.ShapeDtypeStruct((2, _NP), jnp.float32),
    scratch_types=[
        pltpu.VMEM((_DEG_CHUNKS, _C), jnp.int32),
        pltpu.VMEM((_DEG_CHUNKS, _C), jnp.float32),
        pltpu.VMEM((_ROWS_PER_TILE,), jnp.float32),
        pltpu.VMEM_SHARED((_NP,), jnp.float32),
        pltpu.SemaphoreType.DMA,
    ],
    compiler_params=pltpu.CompilerParams(use_tc_tiling_on_sc=False),
)
def _deg_kernel(col_hbm, ew_hbm, out_hbm, col_v, ew_v, zbuf, acc, sem):
    c = lax.axis_index("c")
    s = lax.axis_index("s")
    wid = s * 2 + c

    def _zero(i, carry):
        zbuf[pl.ds(i * 16, 16)] = jnp.zeros((16,), jnp.float32)
        return carry

    lax.fori_loop(0, _ROWS_PER_TILE // 16, _zero, 0)
    pltpu.sync_copy(zbuf, acc.at[pl.ds(s * _ROWS_PER_TILE, _ROWS_PER_TILE)])
    plsc.subcore_barrier()

    base = wid * _DEG_CHUNKS
    pltpu.sync_copy(col_hbm.at[pl.ds(base, _DEG_CHUNKS)], col_v)
    pltpu.sync_copy(ew_hbm.at[pl.ds(base, _DEG_CHUNKS)], ew_v)

    def _chunk(i, carry):
        pltpu.sync_copy(ew_v.at[i], acc.at[col_v.at[i]], add=True)
        return carry

    lax.fori_loop(0, _DEG_CHUNKS, _chunk, 0)
    plsc.subcore_barrier()
    pltpu.sync_copy(acc.at[pl.ds(s * _ROWS_PER_TILE, _ROWS_PER_TILE)],
                    out_hbm.at[c, pl.ds(s * _ROWS_PER_TILE, _ROWS_PER_TILE)])


def _make_scatter_kernel(D, n_tabs):
    ept = _EPAD // 16
    nbuf = 5
    nhalf = 2
    hchunks = _SC_CHUNKS // nhalf
    hedges = ept // nhalf

    @functools.partial(
        pl.kernel,
        mesh=_mesh,
        out_type=[jax.ShapeDtypeStruct((2, _NP, D), jnp.float32)] * n_tabs,
        scratch_types=[
            pltpu.VMEM((hedges,), jnp.int32),
            pltpu.VMEM((hchunks, _C), jnp.int32),
            pltpu.VMEM((hedges,), jnp.float32),
            pltpu.VMEM((nbuf, _C, D), jnp.float32),
            pltpu.VMEM((_C, D), jnp.float32),
            pltpu.VMEM_SHARED((_NP, D), jnp.float32),
            pltpu.SemaphoreType.DMA,
            pltpu.SemaphoreType.DMA,
        ],
        compiler_params=pltpu.CompilerParams(use_tc_tiling_on_sc=False),
    )
    def _scatter(*refs):
        row_hbm, col_hbm, ew_hbm = refs[0:3]
        tabs = refs[3:3 + n_tabs]
        outs = refs[3 + n_tabs:3 + 2 * n_tabs]
        (row_v, col_v, ew_v, bufs, zbuf, acc, gsem,
         ssem) = refs[3 + 2 * n_tabs:]
        c = lax.axis_index("c")
        s = lax.axis_index("s")

        def _zero(i, carry):
            for j in range(D // 16):
                zbuf[i, pl.ds(j * 16, 16)] = jnp.zeros((16,), jnp.float32)
            return carry

        lax.fori_loop(0, _C, _zero, 0)

        def _drain(sem_):
            pltpu.make_async_copy(tabs[0].at[pl.ds(0, _C)], zbuf, sem_).wait()

        def _scale(i, buf):
            def _group(g, gcarry):
                ew16 = ew_v[pl.ds(i * _C + g * 16, 16)]
                e0 = g * 16
                for e16 in range(16):
                    splat = lax.gather(
                        ew16, jnp.full((16, 1), e16, jnp.int32),
                        lax.GatherDimensionNumbers(
                            offset_dims=(), collapsed_slice_dims=(0,),
                            start_index_map=(0,)),
                        slice_sizes=(1,),
                        mode=lax.GatherScatterMode.PROMISE_IN_BOUNDS)
                    e = e0 + e16
                    for j in range(D // 16):
                        buf[e, pl.ds(j * 16, 16)] = (
                            buf[e, pl.ds(j * 16, 16)] * splat)
                return gcarry

            lax.fori_loop(0, _C // 16, _group, 0, unroll=4)

        def _one_pass(tab_hbm, out_hbm):
            for q in range(_ROWS_PER_TILE // _C):
                pltpu.sync_copy(
                    zbuf, acc.at[pl.ds(s * _ROWS_PER_TILE + q * _C, _C)])
            plsc.subcore_barrier()

            for h in range(nhalf):
                pltpu.sync_copy(
                    row_hbm.at[c, pl.ds(s * ept + h * hedges, hedges)], row_v)
                pltpu.sync_copy(
                    col_hbm.at[pl.ds(s * _SC_CHUNKS + h * hchunks, hchunks)],
                    col_v)
                pltpu.sync_copy(
                    ew_hbm.at[pl.ds(s * ept + h * hedges, hedges)], ew_v)

                def _issue_gather(i, b, tab=tab_hbm):
                    pltpu.async_copy(
                        tab.at[row_v.at[pl.ds(i * _C, _C)]], bufs.at[b], gsem)

                _issue_gather(0, 0)
                _issue_gather(1, 1)
                _issue_gather(2, 2)

                def _roll(sp, carry, issue=_issue_gather):
                    for b in range(nbuf):
                        i = sp * nbuf + b
                        _drain(gsem)
                        _scale(i, bufs.at[b])
                        pltpu.async_copy(
                            bufs.at[b], acc.at[col_v.at[i]], ssem, add=True)

                        @pl.when(i >= 2)
                        def _():
                            _drain(ssem)

                        @pl.when(i + 3 < hchunks)
                        def _():
                            issue(i + 3, (b + 3) % nbuf)
                    return carry

                lax.fori_loop(0, hchunks // nbuf, _roll, 0)
                _drain(ssem)
                _drain(ssem)
            plsc.subcore_barrier()
            pltpu.sync_copy(
                acc.at[pl.ds(s * _ROWS_PER_TILE, _ROWS_PER_TILE)],
                out_hbm.at[c, pl.ds(s * _ROWS_PER_TILE, _ROWS_PER_TILE)])

        for tab_hbm, out_hbm in zip(tabs, outs):
            _one_pass(tab_hbm, out_hbm)

    return _scatter


_scatter_dual = _make_scatter_kernel(_DOUT, 2)
_scatter_single = _make_scatter_kernel(_DOUT, 1)


_BM = 2000
_NB = _N // _BM


def _dis_from(degblk):
    deg = degblk[:, 0:1] + degblk[:, 1:2] + 1.0
    return jnp.where(deg > 0, lax.rsqrt(deg), 0.0)


def _xw_body(x_ref, w_ref, deg_ref, ta_ref, tb_ref):
    dis = _dis_from(deg_ref[...])
    t = jnp.dot(x_ref[...], w_ref[...],
                preferred_element_type=jnp.float32) * dis
    ta_ref[...] = t[:, :_DOUT]
    tb_ref[...] = t[:, _DOUT:]


def _layer1_body(tmpa_ref, tmpb_ref, ta_ref, tb_ref, deg_ref, b_ref, a_ref,
                 w2_ref, t2_ref):
    dis = _dis_from(deg_ref[...])
    u = jnp.concatenate(
        [tmpa_ref[0] + ta_ref[...], tmpb_ref[0] + tb_ref[...]],
        axis=1) * dis + b_ref[...]
    u = jnp.where(u >= 0, u, a_ref[...] * u)
    t2_ref[...] = jnp.dot(u, w2_ref[...],
                          preferred_element_type=jnp.float32) * dis


def _layer2_body(tmp_t, tmp_b, t2_t, t2_b, deg_ref, b_ref, a_ref, wd_ref,
                 z_ref, cc_ref, p_ref, pa_ref):
    dis = _dis_from(deg_ref[...])
    zt = (tmp_t[0] + t2_t[...]) * dis + b_ref[...]
    zt = jnp.where(zt >= 0, zt, a_ref[...] * zt)
    zb = (tmp_b[0] + t2_b[...]) * dis + b_ref[...]
    zb = jnp.where(zb >= 0, zb, a_ref[...] * zb)
    z_ref[...] = zt
    cc_ref[...] = jnp.concatenate([zt, zb], axis=1)
    p_ref[...] = jnp.dot(zt, wd_ref[...], preferred_element_type=jnp.float32)
    pa_ref[...] = jnp.dot(zb, wd_ref[...], preferred_element_type=jnp.float32)


_BM5 = 1024
_BK5 = 2048
_NK5 = (_N + _BK5 - 1) // _BK5


def _readout_body(mask_ref, cc_ref, p_ref, pa_ref, bd_ref,
                  ret_ref, reta_ref, g_acc, rs_acc):
    k = pl.program_id(1)
    valid = _N - k * _BK5
    m = jnp.where(
        lax.broadcasted_iota(jnp.int32, (_BM5, _BK5), 1) < valid,
        mask_ref[...], 0.0)
    cc = jnp.where(
        lax.broadcasted_iota(jnp.int32, (_BK5, _DH), 0) < valid,
        cc_ref[...], 0.0)
    part = jnp.dot(m, cc, preferred_element_type=jnp.float32)
    rpart = jnp.sum(m, axis=1, keepdims=True)

    @pl.when(k == 0)
    def _init():
        g_acc[...] = part
        rs_acc[...] = rpart

    @pl.when(k > 0)
    def _accum():
        g_acc[...] = g_acc[...] + part
        rs_acc[...] = rs_acc[...] + rpart

    @pl.when(k == _NK5 - 1)
    def _epilogue():
        avg = g_acc[...] / rs_acc[...]
        ga = avg[:, :_DOUT]
        gb = avg[:, _DOUT:]
        na = jnp.maximum(jnp.sqrt(jnp.sum(ga * ga, axis=1, keepdims=True)),
                         1e-12)
        nb = jnp.maximum(jnp.sqrt(jnp.sum(gb * gb, axis=1, keepdims=True)),
                         1e-12)
        gsa = jax.nn.sigmoid(ga / na)
        gsb = jax.nn.sigmoid(gb / nb)
        p = p_ref[...]
        pa = pa_ref[...]
        bd = bd_ref[0, 0]
        ret_ref[...] = jnp.concatenate(
            [jnp.sum(p * gsa, axis=1, keepdims=True),
             jnp.sum(pa * gsa, axis=1, keepdims=True)], axis=1) + bd
        reta_ref[...] = jnp.concatenate(
            [jnp.sum(pa * gsb, axis=1, keepdims=True),
             jnp.sum(p * gsb, axis=1, keepdims=True)], axis=1) + bd


def kernel(expr, expr_a, edge_list, edge_weight, graph_neigh,
           W1, b1, alpha1, W2, b2, alpha2, Wd, bd):
    f32 = jnp.float32
    row = edge_list[0]
    col = edge_list[1]
    pad = _EPAD - _E
    row_p = jnp.concatenate([row, jnp.zeros((pad,), row.dtype)]).astype(jnp.int32)
    col_p = jnp.concatenate([col, jnp.zeros((pad,), col.dtype)]).astype(jnp.int32)
    ew_p = jnp.concatenate([edge_weight, jnp.zeros((pad,), f32)])
    col2d = col_p.reshape(-1, _C)
    ew2d = ew_p.reshape(-1, _C)
    row2 = jnp.stack([row_p, row_p + _N])

    deg_raw = _deg_kernel(col2d, ew2d)
    degpt = deg_raw.T

    x2 = jnp.concatenate([expr, expr_a], axis=0)

    t1a, t1b = pl.pallas_call(
        _xw_body,
        grid=(2 * _NB,),
        in_specs=[
            pl.BlockSpec((_BM, _DIN), lambda i: (i, 0)),
            pl.BlockSpec((_DIN, _DH), lambda i: (0, 0)),
            pl.BlockSpec((_BM, 2), lambda i: (i % _NB, 0)),
        ],
        out_specs=[
            pl.BlockSpec((_BM, _DOUT), lambda i: (i, 0)),
            pl.BlockSpec((_BM, _DOUT), lambda i: (i, 0)),
        ],
        out_shape=[
            jax.ShapeDtypeStruct((2 * _N, _DOUT), f32),
            jax.ShapeDtypeStruct((2 * _N, _DOUT), f32),
        ],
        compiler_params=pltpu.CompilerParams(
            dimension_semantics=("parallel",)),
    )(x2, W1, degpt)

    tmp1a, tmp1b = _scatter_dual(row2, col2d, ew_p, t1a, t1b)

    t2 = pl.pallas_call(
        _layer1_body,
        grid=(2 * _NB,),
        in_specs=[
            pl.BlockSpec((1, _BM, _DOUT), lambda i: (i // _NB, i % _NB, 0)),
            pl.BlockSpec((1, _BM, _DOUT), lambda i: (i // _NB, i % _NB, 0)),
            pl.BlockSpec((_BM, _DOUT), lambda i: (i, 0)),
            pl.BlockSpec((_BM, _DOUT), lambda i: (i, 0)),
            pl.BlockSpec((_BM, 2), lambda i: (i % _NB, 0)),
            pl.BlockSpec((1, _DH), lambda i: (0, 0)),
            pl.BlockSpec((1, _DH), lambda i: (0, 0)),
            pl.BlockSpec((_DH, _DOUT), lambda i: (0, 0)),
        ],
        out_specs=pl.BlockSpec((_BM, _DOUT), lambda i: (i, 0)),
        out_shape=jax.ShapeDtypeStruct((2 * _N, _DOUT), f32),
        compiler_params=pltpu.CompilerParams(
            dimension_semantics=("parallel",)),
    )(tmp1a, tmp1b, t1a, t1b, degpt, b1.reshape(1, _DH),
      alpha1.reshape(1, _DH), W2)

    tmp2, = _scatter_single(row2, col2d, ew_p, t2)

    z, cc, p, pa = pl.pallas_call(
        _layer2_body,
        grid=(_NB,),
        in_specs=[
            pl.BlockSpec((1, _BM, _DOUT), lambda i: (0, i, 0)),
            pl.BlockSpec((1, _BM, _DOUT), lambda i: (1, i, 0)),
            pl.BlockSpec((_BM, _DOUT), lambda i: (i, 0)),
            pl.BlockSpec((_BM, _DOUT), lambda i: (i + _NB, 0)),
            pl.BlockSpec((_BM, 2), lambda i: (i, 0)),
            pl.BlockSpec((1, _DOUT), lambda i: (0, 0)),
            pl.BlockSpec((1, _DOUT), lambda i: (0, 0)),
            pl.BlockSpec((_DOUT, _DOUT), lambda i: (0, 0)),
        ],
        out_specs=[
            pl.BlockSpec((_BM, _DOUT), lambda i: (i, 0)),
            pl.BlockSpec((_BM, _DH), lambda i: (i, 0)),
            pl.BlockSpec((_BM, _DOUT), lambda i: (i, 0)),
            pl.BlockSpec((_BM, _DOUT), lambda i: (i, 0)),
        ],
        out_shape=[
            jax.ShapeDtypeStruct((_N, _DOUT), f32),
            jax.ShapeDtypeStruct((_N, _DH), f32),
            jax.ShapeDtypeStruct((_N, _DOUT), f32),
            jax.ShapeDtypeStruct((_N, _DOUT), f32),
        ],
        compiler_params=pltpu.CompilerParams(
            dimension_semantics=("parallel",)),
    )(tmp2, tmp2, t2, t2, degpt, b2.reshape(1, _DOUT),
      alpha2.reshape(1, _DOUT), Wd)

    ret, ret_a = pl.pallas_call(
        _readout_body,
        grid=((_N + _BM5 - 1) // _BM5, _NK5),
        in_specs=[
            pl.BlockSpec((_BM5, _BK5), lambda i, k: (i, k)),
            pl.BlockSpec((_BK5, _DH), lambda i, k: (k, 0)),
            pl.BlockSpec((_BM5, _DOUT), lambda i, k: (i, 0)),
            pl.BlockSpec((_BM5, _DOUT), lambda i, k: (i, 0)),
            pl.BlockSpec((1, 1), lambda i, k: (0, 0)),
        ],
        out_specs=[
            pl.BlockSpec((_BM5, 2), lambda i, k: (i, 0)),
            pl.BlockSpec((_BM5, 2), lambda i, k: (i, 0)),
        ],
        out_shape=[
            jax.ShapeDtypeStruct((_N, 2), f32),
            jax.ShapeDtypeStruct((_N, 2), f32),
        ],
        scratch_shapes=[
            pltpu.VMEM((_BM5, _DH), f32),
            pltpu.VMEM((_BM5, 1), f32),
        ],
        compiler_params=pltpu.CompilerParams(
            dimension_semantics=("parallel", "arbitrary")),
    )(graph_neigh, cc, p, pa, bd.reshape(1, 1))

    return (z, ret, ret_a)


# speedup vs baseline: 1.0582x; 1.0105x over previous
import functools

import jax
import jax.numpy as jnp
from jax import lax
from jax.experimental import pallas as pl
from jax.experimental.pallas import tpu as pltpu
from jax.experimental.pallas import tpu_sc as plsc

_N = 10000
_E = 320000
_DIN = 128
_DH = 128
_DOUT = 64
_NP = 10240
_EPAD = 327680
_C = 128
_ROWS_PER_TILE = _NP // 16
_DEG_CHUNKS = _EPAD // (32 * _C)
_SC_CHUNKS = _EPAD // (16 * _C)

_mesh = plsc.VectorSubcoreMesh(core_axis_name="c", subcore_axis_name="s")


@functools.partial(
    pl.kernel,
    mesh=_mesh,
    out_type=jax.ShapeDtypeStruct((2, _NP), jnp.float32),
    scratch_types=[
        pltpu.VMEM((_DEG_CHUNKS, _C), jnp.int32),
        pltpu.VMEM((_DEG_CHUNKS, _C), jnp.float32),
        pltpu.VMEM((_ROWS_PER_TILE,), jnp.float32),
        pltpu.VMEM_SHARED((_NP,), jnp.float32),
        pltpu.SemaphoreType.DMA,
    ],
    compiler_params=pltpu.CompilerParams(use_tc_tiling_on_sc=False),
)
def _deg_kernel(col_hbm, ew_hbm, out_hbm, col_v, ew_v, zbuf, acc, sem):
    c = lax.axis_index("c")
    s = lax.axis_index("s")
    wid = s * 2 + c

    def _zero(i, carry):
        zbuf[pl.ds(i * 16, 16)] = jnp.zeros((16,), jnp.float32)
        return carry

    lax.fori_loop(0, _ROWS_PER_TILE // 16, _zero, 0)
    pltpu.sync_copy(zbuf, acc.at[pl.ds(s * _ROWS_PER_TILE, _ROWS_PER_TILE)])
    plsc.subcore_barrier()

    base = wid * _DEG_CHUNKS
    pltpu.sync_copy(col_hbm.at[pl.ds(base, _DEG_CHUNKS)], col_v)
    pltpu.sync_copy(ew_hbm.at[pl.ds(base, _DEG_CHUNKS)], ew_v)

    def _chunk(i, carry):
        pltpu.sync_copy(ew_v.at[i], acc.at[col_v.at[i]], add=True)
        return carry

    lax.fori_loop(0, _DEG_CHUNKS, _chunk, 0)
    plsc.subcore_barrier()
    pltpu.sync_copy(acc.at[pl.ds(s * _ROWS_PER_TILE, _ROWS_PER_TILE)],
                    out_hbm.at[c, pl.ds(s * _ROWS_PER_TILE, _ROWS_PER_TILE)])


def _make_scatter_kernel(D, n_tabs):
    ept = _EPAD // 16
    nbuf = 5
    nhalf = 2
    hchunks = _SC_CHUNKS // nhalf
    hedges = ept // nhalf

    @functools.partial(
        pl.kernel,
        mesh=_mesh,
        out_type=[jax.ShapeDtypeStruct((2, _NP, D), jnp.float32)] * n_tabs,
        scratch_types=[
            pltpu.VMEM((hedges,), jnp.int32),
            pltpu.VMEM((hchunks, _C), jnp.int32),
            pltpu.VMEM((hedges,), jnp.float32),
            pltpu.VMEM((nbuf, _C, D), jnp.float32),
            pltpu.VMEM((_C, D), jnp.float32),
            pltpu.VMEM_SHARED((_NP, D), jnp.float32),
            pltpu.SemaphoreType.DMA,
            pltpu.SemaphoreType.DMA,
        ],
        compiler_params=pltpu.CompilerParams(use_tc_tiling_on_sc=False),
    )
    def _scatter(*refs):
        row_hbm, col_hbm, ew_hbm = refs[0:3]
        tabs = refs[3:3 + n_tabs]
        outs = refs[3 + n_tabs:3 + 2 * n_tabs]
        (row_v, col_v, ew_v, bufs, zbuf, acc, gsem,
         ssem) = refs[3 + 2 * n_tabs:]
        c = lax.axis_index("c")
        s = lax.axis_index("s")

        def _zero(i, carry):
            for j in range(D // 16):
                zbuf[i, pl.ds(j * 16, 16)] = jnp.zeros((16,), jnp.float32)
            return carry

        lax.fori_loop(0, _C, _zero, 0)

        def _drain(sem_):
            pltpu.make_async_copy(tabs[0].at[pl.ds(0, _C)], zbuf, sem_).wait()

        def _scale(i, buf):
            def _group(g, gcarry):
                ew16 = ew_v[pl.ds(i * _C + g * 16, 16)]
                e0 = g * 16
                for e16 in range(16):
                    splat = lax.gather(
                        ew16, jnp.full((16, 1), e16, jnp.int32),
                        lax.GatherDimensionNumbers(
                            offset_dims=(), collapsed_slice_dims=(0,),
                            start_index_map=(0,)),
                        slice_sizes=(1,),
                        mode=lax.GatherScatterMode.PROMISE_IN_BOUNDS)
                    e = e0 + e16
                    for j in range(D // 16):
                        buf[e, pl.ds(j * 16, 16)] = (
                            buf[e, pl.ds(j * 16, 16)] * splat)
                return gcarry

            lax.fori_loop(0, _C // 16, _group, 0, unroll=4)

        def _one_pass(tab_hbm, out_hbm):
            for q in range(_ROWS_PER_TILE // _C):
                pltpu.sync_copy(
                    zbuf, acc.at[pl.ds(s * _ROWS_PER_TILE + q * _C, _C)])
            plsc.subcore_barrier()

            for h in range(nhalf):
                pltpu.sync_copy(
                    row_hbm.at[c, pl.ds(s * ept + h * hedges, hedges)], row_v)
                pltpu.sync_copy(
                    col_hbm.at[pl.ds(s * _SC_CHUNKS + h * hchunks, hchunks)],
                    col_v)
                pltpu.sync_copy(
                    ew_hbm.at[pl.ds(s * ept + h * hedges, hedges)], ew_v)

                def _issue_gather(i, b, tab=tab_hbm):
                    pltpu.async_copy(
                        tab.at[row_v.at[pl.ds(i * _C, _C)]], bufs.at[b], gsem)

                _issue_gather(0, 0)
                _issue_gather(1, 1)
                _issue_gather(2, 2)

                def _roll(sp, carry, issue=_issue_gather):
                    for b in range(nbuf):
                        i = sp * nbuf + b
                        _drain(gsem)
                        _scale(i, bufs.at[b])
                        pltpu.async_copy(
                            bufs.at[b], acc.at[col_v.at[i]], ssem, add=True)

                        @pl.when(i >= 2)
                        def _():
                            _drain(ssem)

                        @pl.when(i + 3 < hchunks)
                        def _():
                            issue(i + 3, (b + 3) % nbuf)
                    return carry

                lax.fori_loop(0, hchunks // nbuf, _roll, 0)
                _drain(ssem)
                _drain(ssem)
            plsc.subcore_barrier()
            pltpu.sync_copy(
                acc.at[pl.ds(s * _ROWS_PER_TILE, _ROWS_PER_TILE)],
                out_hbm.at[c, pl.ds(s * _ROWS_PER_TILE, _ROWS_PER_TILE)])

        for tab_hbm, out_hbm in zip(tabs, outs):
            _one_pass(tab_hbm, out_hbm)

    return _scatter


_scatter_dual = _make_scatter_kernel(_DOUT, 2)
_scatter_single = _make_scatter_kernel(_DOUT, 1)


_BM = 2000
_NB = _N // _BM


def _dis_from(degblk):
    deg = degblk[:, 0:1] + degblk[:, 1:2] + 1.0
    return jnp.where(deg > 0, lax.rsqrt(deg), 0.0)


def _xw_body(x_ref, w_ref, deg_ref, ta_ref, tb_ref):
    dis = _dis_from(deg_ref[...])
    t = jnp.dot(x_ref[...], w_ref[...],
                preferred_element_type=jnp.float32) * dis
    ta_ref[...] = t[:, :_DOUT]
    tb_ref[...] = t[:, _DOUT:]


def _layer1_body(tmpa_ref, tmpb_ref, ta_ref, tb_ref, deg_ref, b_ref, a_ref,
                 w2_ref, t2_ref):
    dis = _dis_from(deg_ref[...])
    u = jnp.concatenate(
        [tmpa_ref[0] + ta_ref[...], tmpb_ref[0] + tb_ref[...]],
        axis=1) * dis + b_ref[...]
    u = jnp.where(u >= 0, u, a_ref[...] * u)
    t2_ref[...] = jnp.dot(u, w2_ref[...],
                          preferred_element_type=jnp.float32) * dis


def _layer2_body(tmp_t, tmp_b, t2_t, t2_b, deg_ref, b_ref, a_ref, wd_ref,
                 z_ref, cc_ref, p_ref, pa_ref):
    dis = _dis_from(deg_ref[...])
    zt = (tmp_t[0] + t2_t[...]) * dis + b_ref[...]
    zt = jnp.where(zt >= 0, zt, a_ref[...] * zt)
    zb = (tmp_b[0] + t2_b[...]) * dis + b_ref[...]
    zb = jnp.where(zb >= 0, zb, a_ref[...] * zb)
    z_ref[...] = zt
    cc_ref[...] = jnp.concatenate([zt, zb], axis=1)
    p_ref[...] = jnp.dot(zt, wd_ref[...], preferred_element_type=jnp.float32)
    pa_ref[...] = jnp.dot(zb, wd_ref[...], preferred_element_type=jnp.float32)


_BM5 = 1024
_BK5 = 2560
_NK5 = (_N + _BK5 - 1) // _BK5


def _readout_body(mask_ref, cc_ref, p_ref, pa_ref, bd_ref,
                  ret_ref, reta_ref, g_acc, rs_acc):
    k = pl.program_id(1)
    valid = _N - k * _BK5
    m = jnp.where(
        lax.broadcasted_iota(jnp.int32, (_BM5, _BK5), 1) < valid,
        mask_ref[...], 0.0)
    cc = jnp.where(
        lax.broadcasted_iota(jnp.int32, (_BK5, _DH), 0) < valid,
        cc_ref[...], 0.0)
    part = jnp.dot(m, cc, preferred_element_type=jnp.float32)
    rpart = jnp.sum(m, axis=1, keepdims=True)

    @pl.when(k == 0)
    def _init():
        g_acc[...] = part
        rs_acc[...] = rpart

    @pl.when(k > 0)
    def _accum():
        g_acc[...] = g_acc[...] + part
        rs_acc[...] = rs_acc[...] + rpart

    @pl.when(k == _NK5 - 1)
    def _epilogue():
        avg = g_acc[...] / rs_acc[...]
        ga = avg[:, :_DOUT]
        gb = avg[:, _DOUT:]
        na = jnp.maximum(jnp.sqrt(jnp.sum(ga * ga, axis=1, keepdims=True)),
                         1e-12)
        nb = jnp.maximum(jnp.sqrt(jnp.sum(gb * gb, axis=1, keepdims=True)),
                         1e-12)
        gsa = jax.nn.sigmoid(ga / na)
        gsb = jax.nn.sigmoid(gb / nb)
        p = p_ref[...]
        pa = pa_ref[...]
        bd = bd_ref[0, 0]
        ret_ref[...] = jnp.concatenate(
            [jnp.sum(p * gsa, axis=1, keepdims=True),
             jnp.sum(pa * gsa, axis=1, keepdims=True)], axis=1) + bd
        reta_ref[...] = jnp.concatenate(
            [jnp.sum(pa * gsb, axis=1, keepdims=True),
             jnp.sum(p * gsb, axis=1, keepdims=True)], axis=1) + bd


def kernel(expr, expr_a, edge_list, edge_weight, graph_neigh,
           W1, b1, alpha1, W2, b2, alpha2, Wd, bd):
    f32 = jnp.float32
    row = edge_list[0]
    col = edge_list[1]
    pad = _EPAD - _E
    row_p = jnp.concatenate([row, jnp.zeros((pad,), row.dtype)]).astype(jnp.int32)
    col_p = jnp.concatenate([col, jnp.zeros((pad,), col.dtype)]).astype(jnp.int32)
    ew_p = jnp.concatenate([edge_weight, jnp.zeros((pad,), f32)])
    col2d = col_p.reshape(-1, _C)
    ew2d = ew_p.reshape(-1, _C)
    row2 = jnp.stack([row_p, row_p + _N])

    deg_raw = _deg_kernel(col2d, ew2d)
    degpt = deg_raw.T

    x2 = jnp.concatenate([expr, expr_a], axis=0)

    t1a, t1b = pl.pallas_call(
        _xw_body,
        grid=(2 * _NB,),
        in_specs=[
            pl.BlockSpec((_BM, _DIN), lambda i: (i, 0)),
            pl.BlockSpec((_DIN, _DH), lambda i: (0, 0)),
            pl.BlockSpec((_BM, 2), lambda i: (i % _NB, 0)),
        ],
        out_specs=[
            pl.BlockSpec((_BM, _DOUT), lambda i: (i, 0)),
            pl.BlockSpec((_BM, _DOUT), lambda i: (i, 0)),
        ],
        out_shape=[
            jax.ShapeDtypeStruct((2 * _N, _DOUT), f32),
            jax.ShapeDtypeStruct((2 * _N, _DOUT), f32),
        ],
        compiler_params=pltpu.CompilerParams(
            dimension_semantics=("parallel",)),
    )(x2, W1, degpt)

    tmp1a, tmp1b = _scatter_dual(row2, col2d, ew_p, t1a, t1b)

    t2 = pl.pallas_call(
        _layer1_body,
        grid=(2 * _NB,),
        in_specs=[
            pl.BlockSpec((1, _BM, _DOUT), lambda i: (i // _NB, i % _NB, 0)),
            pl.BlockSpec((1, _BM, _DOUT), lambda i: (i // _NB, i % _NB, 0)),
            pl.BlockSpec((_BM, _DOUT), lambda i: (i, 0)),
            pl.BlockSpec((_BM, _DOUT), lambda i: (i, 0)),
            pl.BlockSpec((_BM, 2), lambda i: (i % _NB, 0)),
            pl.BlockSpec((1, _DH), lambda i: (0, 0)),
            pl.BlockSpec((1, _DH), lambda i: (0, 0)),
            pl.BlockSpec((_DH, _DOUT), lambda i: (0, 0)),
        ],
        out_specs=pl.BlockSpec((_BM, _DOUT), lambda i: (i, 0)),
        out_shape=jax.ShapeDtypeStruct((2 * _N, _DOUT), f32),
        compiler_params=pltpu.CompilerParams(
            dimension_semantics=("parallel",)),
    )(tmp1a, tmp1b, t1a, t1b, degpt, b1.reshape(1, _DH),
      alpha1.reshape(1, _DH), W2)

    tmp2, = _scatter_single(row2, col2d, ew_p, t2)

    z, cc, p, pa = pl.pallas_call(
        _layer2_body,
        grid=(_NB,),
        in_specs=[
            pl.BlockSpec((1, _BM, _DOUT), lambda i: (0, i, 0)),
            pl.BlockSpec((1, _BM, _DOUT), lambda i: (1, i, 0)),
            pl.BlockSpec((_BM, _DOUT), lambda i: (i, 0)),
            pl.BlockSpec((_BM, _DOUT), lambda i: (i + _NB, 0)),
            pl.BlockSpec((_BM, 2), lambda i: (i, 0)),
            pl.BlockSpec((1, _DOUT), lambda i: (0, 0)),
            pl.BlockSpec((1, _DOUT), lambda i: (0, 0)),
            pl.BlockSpec((_DOUT, _DOUT), lambda i: (0, 0)),
        ],
        out_specs=[
            pl.BlockSpec((_BM, _DOUT), lambda i: (i, 0)),
            pl.BlockSpec((_BM, _DH), lambda i: (i, 0)),
            pl.BlockSpec((_BM, _DOUT), lambda i: (i, 0)),
            pl.BlockSpec((_BM, _DOUT), lambda i: (i, 0)),
        ],
        out_shape=[
            jax.ShapeDtypeStruct((_N, _DOUT), f32),
            jax.ShapeDtypeStruct((_N, _DH), f32),
            jax.ShapeDtypeStruct((_N, _DOUT), f32),
            jax.ShapeDtypeStruct((_N, _DOUT), f32),
        ],
        compiler_params=pltpu.CompilerParams(
            dimension_semantics=("parallel",)),
    )(tmp2, tmp2, t2, t2, degpt, b2.reshape(1, _DOUT),
      alpha2.reshape(1, _DOUT), Wd)

    ret, ret_a = pl.pallas_call(
        _readout_body,
        grid=((_N + _BM5 - 1) // _BM5, _NK5),
        in_specs=[
            pl.BlockSpec((_BM5, _BK5), lambda i, k: (i, k)),
            pl.BlockSpec((_BK5, _DH), lambda i, k: (k, 0)),
            pl.BlockSpec((_BM5, _DOUT), lambda i, k: (i, 0)),
            pl.BlockSpec((_BM5, _DOUT), lambda i, k: (i, 0)),
            pl.BlockSpec((1, 1), lambda i, k: (0, 0)),
        ],
        out_specs=[
            pl.BlockSpec((_BM5, 2), lambda i, k: (i, 0)),
            pl.BlockSpec((_BM5, 2), lambda i, k: (i, 0)),
        ],
        out_shape=[
            jax.ShapeDtypeStruct((_N, 2), f32),
            jax.ShapeDtypeStruct((_N, 2), f32),
        ],
        scratch_shapes=[
            pltpu.VMEM((_BM5, _DH), f32),
            pltpu.VMEM((_BM5, 1), f32),
        ],
        compiler_params=pltpu.CompilerParams(
            dimension_semantics=("parallel", "arbitrary")),
    )(graph_neigh, cc, p, pa, bd.reshape(1, 1))

    return (z, ret, ret_a)
